# baseline (speedup 1.0000x reference)
import functools

import jax
import jax.numpy as jnp
from jax import lax
from jax.experimental import pallas as pl
from jax.experimental.pallas import tpu as pltpu
from jax.experimental.pallas import tpu_sc as plsc

N = 10000
NP = 10240
E = 320000
HID = 128
HW = HID // 2
NT0, NT1, NT2 = 4000, 3000, 3000
CLUSTERS = 4
NUM_CLASSES = 8

NC, NS = 2, 16
EPS = E // NS
C = 128
EPSP = 20480
NCHUNK = EPSP // C
EPTP = 10240
NCH16 = EPTP // C
NPH = 2
PH = NCHUNK // NPH
NBUF = 5
LOOK = 2
RPT = NP // NS
DEGW = 16


@functools.lru_cache(maxsize=None)
def _sc_agg(with_deg):
  mesh = plsc.VectorSubcoreMesh(core_axis_name="c", subcore_axis_name="s")
  part_t = jax.ShapeDtypeStruct((NC * NP, HW), jnp.float32)
  if with_deg:
    out_type = (part_t, jax.ShapeDtypeStruct((NP, DEGW), jnp.float32))
  else:
    out_type = part_t
  scratch = [
      pltpu.VMEM((PH, C), jnp.int32),
      pltpu.VMEM((PH, C), jnp.int32),
      pltpu.VMEM((NBUF, C, HW), jnp.float32),
      pltpu.VMEM((C, DEGW), jnp.float32),
      pltpu.VMEM_SHARED((NP, HW), jnp.float32),
      pltpu.VMEM_SHARED((NP, DEGW), jnp.float32),
      [pltpu.SemaphoreType.DMA] * NBUF,
      [pltpu.SemaphoreType.DMA] * NBUF,
      [pltpu.SemaphoreType.DMA] * NBUF,
  ]

  @functools.partial(pl.kernel, mesh=mesh, out_type=out_type,
                     scratch_types=scratch,
                     compiler_params=pltpu.CompilerParams(
                         use_tc_tiling_on_sc=False))
  def k(table, src2, dst2, *refs):
    if with_deg:
      out, deg_out, src_a, dst_a, rows, ones_v, acc, dega, gsem, ssem, dsem = refs
    else:
      out, src_a, dst_a, rows, ones_v, acc, dega, gsem, ssem, dsem = refs
      deg_out = None
    c = lax.axis_index("c")
    s = lax.axis_index("s")
    base_r = s * RPT
    srow = c * NP

    def _zero_rows(i, _):
      for j in range(HW // 16):
        rows[0, i, pl.ds(j * 16, 16)] = jnp.zeros((16,), jnp.float32)
      ones_v[i] = jnp.zeros((DEGW,), jnp.float32)
      return 0
    lax.fori_loop(0, C, _zero_rows, 0)

    for z in range(RPT // C):
      pltpu.sync_copy(rows.at[0], acc.at[pl.ds(base_r + z * C, C)])
      if with_deg:
        pltpu.sync_copy(ones_v, dega.at[pl.ds(base_r + z * C, C)])
    plsc.subcore_barrier()

    if with_deg:
      def _fill_ones(i, _):
        ones_v[i] = jnp.ones((DEGW,), jnp.float32)
        return 0
      lax.fori_loop(0, C, _fill_ones, 0)

    def _fire_gather(g, b):
      pltpu.async_copy(table.at[src_a.at[g]], rows.at[b], gsem[b])

    for m in range(NPH):
      pltpu.sync_copy(src2.at[pl.ds(s * NCHUNK + m * PH, PH)], src_a)
      pltpu.sync_copy(dst2.at[pl.ds(s * NCHUNK + m * PH, PH)], dst_a)

      @pl.when(c == 1)
      def _():
        def _adj(i, _):
          for j in range(C // 16):
            sl = pl.ds(j * 16, 16)
            src_a[i, sl] = src_a[i, sl] + NP
          return 0
        lax.fori_loop(0, PH, _adj, 0)

      for b in range(LOOK):
        _fire_gather(b, b)

      def _outer(o, _):
        for b in range(NBUF):
          g = o * NBUF + b
          pltpu.make_async_copy(table.at[src_a.at[g]], rows.at[b],
                                gsem[b]).wait()
          pltpu.async_copy(rows.at[b], acc.at[dst_a.at[g]], ssem[b], add=True)
          if with_deg:
            @pl.when(c == 0)
            def _():
              pltpu.async_copy(ones_v, dega.at[dst_a.at[g]], dsem[b], add=True)
          gn = g + LOOK
          bn = (b + LOOK) % NBUF

          @pl.when(gn < PH)
          def _():
            @pl.when(gn >= NBUF)
            def _():
              pltpu.make_async_copy(rows.at[bn], acc.at[dst_a.at[gn - NBUF]],
                                    ssem[bn]).wait()
              if with_deg:
                @pl.when(c == 0)
                def _():
                  pltpu.make_async_copy(ones_v, dega.at[dst_a.at[gn - NBUF]],
                                        dsem[bn]).wait()
            _fire_gather(gn, bn)
        return 0
      lax.fori_loop(0, PH // NBUF, _outer, 0)

      for b in range(NBUF):
        g = PH - NBUF + b
        pltpu.make_async_copy(rows.at[b], acc.at[dst_a.at[g]], ssem[b]).wait()
        if with_deg:
          @pl.when(c == 0)
          def _():
            pltpu.make_async_copy(ones_v, dega.at[dst_a.at[g]],
                                  dsem[b]).wait()

    plsc.subcore_barrier()
    pltpu.sync_copy(acc.at[pl.ds(base_r, RPT)],
                    out.at[pl.ds(srow + base_r, RPT)])
    if with_deg:
      @pl.when(c == 0)
      def _():
        pltpu.sync_copy(dega.at[pl.ds(base_r, RPT)],
                        deg_out.at[pl.ds(base_r, RPT)])

  return k


@functools.lru_cache(maxsize=None)
def _sc_agg16():
  mesh = plsc.VectorSubcoreMesh(core_axis_name="c", subcore_axis_name="s")
  out_type = jax.ShapeDtypeStruct((NC * NP, DEGW), jnp.float32)
  NCH = NCH16
  scratch = [
      pltpu.VMEM((NCH, C), jnp.int32),
      pltpu.VMEM((NCH, C), jnp.int32),
      pltpu.VMEM((NBUF, C, DEGW), jnp.float32),
      pltpu.VMEM_SHARED((NP, DEGW), jnp.float32),
      [pltpu.SemaphoreType.DMA] * NBUF,
      [pltpu.SemaphoreType.DMA] * NBUF,
  ]

  @functools.partial(pl.kernel, mesh=mesh, out_type=out_type,
                     scratch_types=scratch,
                     compiler_params=pltpu.CompilerParams(
                         use_tc_tiling_on_sc=False))
  def k(table, src2, dst2, out, src_a, dst_a, rows, acc, gsem, ssem):
    c = lax.axis_index("c")
    s = lax.axis_index("s")
    t = c * NS + s
    base_r = s * RPT

    pltpu.sync_copy(src2.at[pl.ds(t * NCH, NCH)], src_a)
    pltpu.sync_copy(dst2.at[pl.ds(t * NCH, NCH)], dst_a)

    def _zero_rows(i, _):
      rows[0, i] = jnp.zeros((DEGW,), jnp.float32)
      return 0
    lax.fori_loop(0, C, _zero_rows, 0)
    for z in range(RPT // C):
      pltpu.sync_copy(rows.at[0], acc.at[pl.ds(base_r + z * C, C)])
    plsc.subcore_barrier()

    def _fire_gather(g, b):
      pltpu.async_copy(table.at[src_a.at[g]], rows.at[b], gsem[b])

    for b in range(LOOK):
      _fire_gather(b, b)

    def _outer(o, _):
      for b in range(NBUF):
        g = o * NBUF + b
        pltpu.make_async_copy(table.at[src_a.at[g]], rows.at[b],
                              gsem[b]).wait()
        pltpu.async_copy(rows.at[b], acc.at[dst_a.at[g]], ssem[b], add=True)
        gn = g + LOOK
        bn = (b + LOOK) % NBUF

        @pl.when(gn < NCH)
        def _():
          @pl.when(gn >= NBUF)
          def _():
            pltpu.make_async_copy(rows.at[bn], acc.at[dst_a.at[gn - NBUF]],
                                  ssem[bn]).wait()
          _fire_gather(gn, bn)
      return 0
    lax.fori_loop(0, NCH // NBUF, _outer, 0)

    for b in range(NBUF):
      g = NCH - NBUF + b
      pltpu.make_async_copy(rows.at[b], acc.at[dst_a.at[g]], ssem[b]).wait()

    plsc.subcore_barrier()
    pltpu.sync_copy(acc.at[pl.ds(base_r, RPT)],
                    out.at[pl.ds(c * NP + base_r, RPT)])

  return k


def _tc_prepare(features, W_pre, b_pre2, W_emb1, b_emb1_2, W_emb2, b_emb2_2):
  def body(f_ref, wp_ref, bp_ref, w1_ref, b1_ref, w2_ref, b2_ref,
           table_ref, base_ref):
    h = jnp.dot(f_ref[...], wp_ref[...],
                preferred_element_type=jnp.float32) + bp_ref[...]
    hp = jnp.concatenate([h, jnp.zeros((NP - NT0, HID), jnp.float32)], axis=0)
    table_ref[...] = jnp.concatenate([hp[:, :HW], hp[:, HW:]], axis=0)
    base_ref[...] = jnp.concatenate(
        [h, w1_ref[...] + b1_ref[...], w2_ref[...] + b2_ref[...],
         jnp.zeros((NP - N, HID), jnp.float32)], axis=0)

  return pl.pallas_call(
      body,
      out_shape=(jax.ShapeDtypeStruct((NC * NP, HW), jnp.float32),
                 jax.ShapeDtypeStruct((NP, HID), jnp.float32)),
  )(features, W_pre, b_pre2, W_emb1, b_emb1_2, W_emb2, b_emb2_2)


def _tc_mid(part0, degp, base, assign2, W_ops, W_out_p):
  def body(p_ref, d_ref, b_ref, a_ref, w_ref, wo_ref, ne_ref, z_ref):
    p = p_ref[...]
    acc = jnp.concatenate([p[:NP], p[NP:]], axis=1)
    deg = jnp.maximum(d_ref[...][:, 0:1], 1.0)
    agg0 = acc / deg
    a = a_ref[...]
    w = w_ref[...]
    t = jnp.zeros((NP, HID), jnp.float32)
    for k in range(CLUSTERS):
      mk = (a == k).astype(jnp.float32)
      t = t + mk * jnp.dot(agg0, w[k], preferred_element_type=jnp.float32)
    h = b_ref[...] + t
    ne = jnp.where(h > 0, h, jnp.exp(h) - 1.0)
    ne_ref[...] = ne
    z_ref[...] = jnp.dot(ne, wo_ref[...], preferred_element_type=jnp.float32)

  return pl.pallas_call(
      body,
      out_shape=(jax.ShapeDtypeStruct((NP, HID), jnp.float32),
                 jax.ShapeDtypeStruct((NP, DEGW), jnp.float32)),
  )(part0, degp, base, assign2, W_ops, W_out_p)


def _tc_head(part1, degp, b_out_p2):
  def body(p_ref, d_ref, b_ref, out_ref):
    p = p_ref[...]
    acc = p[:NP] + p[NP:]
    deg = jnp.maximum(d_ref[...][:, 0:1], 1.0)
    out_ref[...] = acc / deg + b_ref[...]

  return pl.pallas_call(
      body,
      out_shape=jax.ShapeDtypeStruct((NP, DEGW), jnp.float32),
  )(part1, degp, b_out_p2)


def kernel(features, edge_index, node_assign, W_pre, b_pre, W_emb1, b_emb1,
           W_emb2, b_emb2, W_ops, W_out, b_out):
  src = edge_index[0].astype(jnp.int32)
  dst = edge_index[1].astype(jnp.int32)
  pad_a = EPSP - E // NS
  srcA = jnp.pad(src.reshape(NS, E // NS), ((0, 0), (0, pad_a)),
                 constant_values=NT0).reshape(NS * NCHUNK, C)
  dstA = jnp.pad(dst.reshape(NS, E // NS), ((0, 0), (0, pad_a)),
                 constant_values=NP - 1).reshape(NS * NCHUNK, C)
  pad_b = EPTP - E // (NC * NS)
  srcB = jnp.pad(src.reshape(NC * NS, E // (NC * NS)), ((0, 0), (0, pad_b)),
                 constant_values=0).reshape(NC * NS * NCH16, C)
  dstB = jnp.pad(dst.reshape(NC * NS, E // (NC * NS)), ((0, 0), (0, pad_b)),
                 constant_values=NP - 1).reshape(NC * NS * NCH16, C)
  table, base = _tc_prepare(
      features, W_pre, b_pre.reshape(1, HID),
      W_emb1, b_emb1.reshape(1, HID), W_emb2, b_emb2.reshape(1, HID))
  part0, degp = _sc_agg(True)(table, srcA, dstA)
  assign2 = jnp.pad(node_assign.astype(jnp.int32), (0, NP - N),
                    constant_values=-1).reshape(NP, 1)
  W_out_p = jnp.pad(W_out, ((0, 0), (0, DEGW - NUM_CLASSES)))
  ne_p, z = _tc_mid(part0, degp, base, assign2, W_ops, W_out_p)
  part1 = _sc_agg16()(z, srcB, dstB)
  b_out_p = jnp.pad(b_out, (0, DEGW - NUM_CLASSES)).reshape(1, DEGW)
  logits_p = _tc_head(part1, degp, b_out_p)
  return ne_p[:N], logits_p[:N, :NUM_CLASSES]


# speedup vs baseline: 1.4484x; 1.4484x over previous
import functools

import jax
import jax.numpy as jnp
from jax import lax
from jax.experimental import pallas as pl
from jax.experimental.pallas import tpu as pltpu
from jax.experimental.pallas import tpu_sc as plsc

N = 10000
NP = 10240
E = 320000
HID = 128
HW = HID // 2
HWD = HW + 16
NT0, NT1, NT2 = 4000, 3000, 3000
CLUSTERS = 4
NUM_CLASSES = 8

NC, NS = 2, 16
EPS = E // NS
C = 80
NCHUNK = EPS // C
NBUF = 5
LOOK = 2
RPT = NP // NS
DEGW = 16


@functools.lru_cache(maxsize=None)
def _sc_agg80():
  mesh = plsc.VectorSubcoreMesh(core_axis_name="c", subcore_axis_name="s")
  out_type = jax.ShapeDtypeStruct((NC * NP, HWD), jnp.float32)
  scratch = [
      pltpu.VMEM((NCHUNK, C), jnp.int32),
      pltpu.VMEM((NCHUNK, C), jnp.int32),
      pltpu.VMEM((NBUF, C, HWD), jnp.float32),
      pltpu.VMEM_SHARED((NP, HWD), jnp.float32),
      [pltpu.SemaphoreType.DMA] * NBUF,
      [pltpu.SemaphoreType.DMA] * NBUF,
  ]

  @functools.partial(pl.kernel, mesh=mesh, out_type=out_type,
                     scratch_types=scratch,
                     compiler_params=pltpu.CompilerParams(
                         use_tc_tiling_on_sc=False))
  def k(table, src2, dst2, out, src_a, dst_a, rows, acc, gsem, ssem):
    c = lax.axis_index("c")
    s = lax.axis_index("s")
    base_r = s * RPT
    srow = c * NP

    pltpu.sync_copy(src2.at[pl.ds(s * NCHUNK, NCHUNK)], src_a)
    pltpu.sync_copy(dst2.at[pl.ds(s * NCHUNK, NCHUNK)], dst_a)

    @pl.when(c == 1)
    def _():
      def _adj(i, _):
        for j in range(C // 16):
          sl = pl.ds(j * 16, 16)
          src_a[i, sl] = src_a[i, sl] + NP
        return 0
      lax.fori_loop(0, NCHUNK, _adj, 0)

    def _zero_rows(i, _):
      for j in range(HWD // 16):
        rows[0, i, pl.ds(j * 16, 16)] = jnp.zeros((16,), jnp.float32)
      return 0
    lax.fori_loop(0, C, _zero_rows, 0)

    for z in range(RPT // C):
      pltpu.sync_copy(rows.at[0], acc.at[pl.ds(base_r + z * C, C)])
    plsc.subcore_barrier()

    def _fire_gather(g, b):
      pltpu.async_copy(table.at[src_a.at[g]], rows.at[b], gsem[b])

    for b in range(LOOK):
      _fire_gather(b, b)

    def _outer(o, _):
      for b in range(NBUF):
        g = o * NBUF + b
        pltpu.make_async_copy(table.at[src_a.at[g]], rows.at[b],
                              gsem[b]).wait()
        pltpu.async_copy(rows.at[b], acc.at[dst_a.at[g]], ssem[b], add=True)
        gn = g + LOOK
        bn = (b + LOOK) % NBUF

        @pl.when(gn < NCHUNK)
        def _():
          @pl.when(gn >= NBUF)
          def _():
            pltpu.make_async_copy(rows.at[bn], acc.at[dst_a.at[gn - NBUF]],
                                  ssem[bn]).wait()
          _fire_gather(gn, bn)
      return 0
    lax.fori_loop(0, NCHUNK // NBUF, _outer, 0)

    for b in range(NBUF):
      g = NCHUNK - NBUF + b
      pltpu.make_async_copy(rows.at[b], acc.at[dst_a.at[g]], ssem[b]).wait()

    plsc.subcore_barrier()
    pltpu.sync_copy(acc.at[pl.ds(base_r, RPT)],
                    out.at[pl.ds(srow + base_r, RPT)])

  return k


@functools.lru_cache(maxsize=None)
def _sc_agg16():
  mesh = plsc.VectorSubcoreMesh(core_axis_name="c", subcore_axis_name="s")
  out_type = jax.ShapeDtypeStruct((NC * NP, DEGW), jnp.float32)
  NCH = E // (NC * NS) // C
  scratch = [
      pltpu.VMEM((NCH, C), jnp.int32),
      pltpu.VMEM((NCH, C), jnp.int32),
      pltpu.VMEM((NBUF, C, DEGW), jnp.float32),
      pltpu.VMEM_SHARED((NP, DEGW), jnp.float32),
      [pltpu.SemaphoreType.DMA] * NBUF,
      [pltpu.SemaphoreType.DMA] * NBUF,
  ]

  @functools.partial(pl.kernel, mesh=mesh, out_type=out_type,
                     scratch_types=scratch,
                     compiler_params=pltpu.CompilerParams(
                         use_tc_tiling_on_sc=False))
  def k(table, src2, dst2, out, src_a, dst_a, rows, acc, gsem, ssem):
    c = lax.axis_index("c")
    s = lax.axis_index("s")
    t = c * NS + s
    base_r = s * RPT

    pltpu.sync_copy(src2.at[pl.ds(t * NCH, NCH)], src_a)
    pltpu.sync_copy(dst2.at[pl.ds(t * NCH, NCH)], dst_a)

    def _zero_rows(i, _):
      rows[0, i] = jnp.zeros((DEGW,), jnp.float32)
      return 0
    lax.fori_loop(0, C, _zero_rows, 0)
    for z in range(RPT // C):
      pltpu.sync_copy(rows.at[0], acc.at[pl.ds(base_r + z * C, C)])
    plsc.subcore_barrier()

    def _fire_gather(g, b):
      pltpu.async_copy(table.at[src_a.at[g]], rows.at[b], gsem[b])

    for b in range(LOOK):
      _fire_gather(b, b)

    def _outer(o, _):
      for b in range(NBUF):
        g = o * NBUF + b
        pltpu.make_async_copy(table.at[src_a.at[g]], rows.at[b],
                              gsem[b]).wait()
        pltpu.async_copy(rows.at[b], acc.at[dst_a.at[g]], ssem[b], add=True)
        gn = g + LOOK
        bn = (b + LOOK) % NBUF

        @pl.when(gn < NCH)
        def _():
          @pl.when(gn >= NBUF)
          def _():
            pltpu.make_async_copy(rows.at[bn], acc.at[dst_a.at[gn - NBUF]],
                                  ssem[bn]).wait()
          _fire_gather(gn, bn)
      return 0
    lax.fori_loop(0, NCH // NBUF, _outer, 0)

    for b in range(NBUF):
      g = NCH - NBUF + b
      pltpu.make_async_copy(rows.at[b], acc.at[dst_a.at[g]], ssem[b]).wait()

    plsc.subcore_barrier()
    pltpu.sync_copy(acc.at[pl.ds(base_r, RPT)],
                    out.at[pl.ds(c * NP + base_r, RPT)])

  return k


def _tc_prepare(features, W_pre, b_pre2, W_emb1, b_emb1_2, W_emb2, b_emb2_2):
  def body(f_ref, wp_ref, bp_ref, w1_ref, b1_ref, w2_ref, b2_ref,
           table_ref, base_ref):
    h = jnp.dot(f_ref[...], wp_ref[...],
                preferred_element_type=jnp.float32) + bp_ref[...]
    hp = jnp.concatenate([h, jnp.zeros((NP - NT0, HID), jnp.float32)], axis=0)
    top = jnp.concatenate(
        [hp[:, :HW], jnp.zeros((NP, HWD - HW), jnp.float32)], axis=1)
    bot = jnp.concatenate(
        [hp[:, HW:], jnp.ones((NP, HWD - HW), jnp.float32)], axis=1)
    table_ref[...] = jnp.concatenate([top, bot], axis=0)
    base_ref[...] = jnp.concatenate(
        [h, w1_ref[...] + b1_ref[...], w2_ref[...] + b2_ref[...],
         jnp.zeros((NP - N, HID), jnp.float32)], axis=0)

  return pl.pallas_call(
      body,
      out_shape=(jax.ShapeDtypeStruct((NC * NP, HWD), jnp.float32),
                 jax.ShapeDtypeStruct((NP, HID), jnp.float32)),
  )(features, W_pre, b_pre2, W_emb1, b_emb1_2, W_emb2, b_emb2_2)


def _tc_mid(part0, base, assign2, W_ops, W_out_p):
  def body(p_ref, b_ref, a_ref, w_ref, wo_ref, ne_ref, z_ref, dw_ref):
    p = p_ref[...]
    acc = jnp.concatenate([p[:NP, :HW], p[NP:, :HW]], axis=1)
    deg = jnp.maximum(p[NP:, HW:HW + 1], 1.0)
    agg0 = acc / deg
    a = a_ref[...]
    w = w_ref[...]
    t = jnp.zeros((NP, HID), jnp.float32)
    for k in range(CLUSTERS):
      mk = (a == k).astype(jnp.float32)
      t = t + mk * jnp.dot(agg0, w[k], preferred_element_type=jnp.float32)
    h = b_ref[...] + t
    ne = jnp.where(h > 0, h, jnp.exp(h) - 1.0)
    ne_ref[...] = ne
    z_ref[...] = jnp.dot(ne, wo_ref[...], preferred_element_type=jnp.float32)
    dw_ref[...] = jnp.broadcast_to(deg, (NP, DEGW))

  return pl.pallas_call(
      body,
      out_shape=(jax.ShapeDtypeStruct((NP, HID), jnp.float32),
                 jax.ShapeDtypeStruct((NP, DEGW), jnp.float32),
                 jax.ShapeDtypeStruct((NP, DEGW), jnp.float32)),
  )(part0, base, assign2, W_ops, W_out_p)


def _tc_head(part1, degw, b_out_p2):
  def body(p_ref, d_ref, b_ref, out_ref):
    p = p_ref[...]
    acc = p[:NP] + p[NP:]
    deg = d_ref[...][:, 0:1]
    out_ref[...] = acc / deg + b_ref[...]

  return pl.pallas_call(
      body,
      out_shape=jax.ShapeDtypeStruct((NP, DEGW), jnp.float32),
  )(part1, degw, b_out_p2)


def kernel(features, edge_index, node_assign, W_pre, b_pre, W_emb1, b_emb1,
           W_emb2, b_emb2, W_ops, W_out, b_out):
  src = edge_index[0].astype(jnp.int32).reshape(E // C, C)
  dst = edge_index[1].astype(jnp.int32).reshape(E // C, C)
  table, base = _tc_prepare(
      features, W_pre, b_pre.reshape(1, HID),
      W_emb1, b_emb1.reshape(1, HID), W_emb2, b_emb2.reshape(1, HID))
  part0 = _sc_agg80()(table, src, dst)
  assign2 = jnp.pad(node_assign.astype(jnp.int32), (0, NP - N),
                    constant_values=-1).reshape(NP, 1)
  W_out_p = jnp.pad(W_out, ((0, 0), (0, DEGW - NUM_CLASSES)))
  ne_p, z, degw = _tc_mid(part0, base, assign2, W_ops, W_out_p)
  part1 = _sc_agg16()(z, src, dst)
  b_out_p = jnp.pad(b_out, (0, DEGW - NUM_CLASSES)).reshape(1, DEGW)
  logits_p = _tc_head(part1, degw, b_out_p)
  return ne_p[:N], logits_p[:N, :NUM_CLASSES]


# speedup vs baseline: 1.8702x; 1.2912x over previous
import functools

import jax
import jax.numpy as jnp
from jax import lax
from jax.experimental import pallas as pl
from jax.experimental.pallas import tpu as pltpu
from jax.experimental.pallas import tpu_sc as plsc

N = 10000
NP = 10240
E = 320000
HID = 128
HW = HID // 2
NT0, NT1, NT2 = 4000, 3000, 3000
CLUSTERS = 4
NUM_CLASSES = 8

NC, NS = 2, 16
EPS = E // NS
C = 80
NCHUNK = EPS // C
NBUF = 5
LOOK = 4
RPT = NP // NS
DEGW = 16


@functools.lru_cache(maxsize=None)
def _sc_agg(with_deg):
  mesh = plsc.VectorSubcoreMesh(core_axis_name="c", subcore_axis_name="s")
  part_t = jax.ShapeDtypeStruct((NC * NP, HW), jnp.float32)
  if with_deg:
    out_type = (part_t, jax.ShapeDtypeStruct((NP, DEGW), jnp.float32))
  else:
    out_type = part_t
  scratch = [
      pltpu.VMEM((NCHUNK, C), jnp.int32),
      pltpu.VMEM((NCHUNK, C), jnp.int32),
      pltpu.VMEM((NBUF, C, HW), jnp.float32),
      pltpu.VMEM((C, DEGW), jnp.float32),
      pltpu.VMEM_SHARED((NP, HW), jnp.float32),
      pltpu.VMEM_SHARED((NP, DEGW), jnp.float32),
      [pltpu.SemaphoreType.DMA] * NBUF,
      [pltpu.SemaphoreType.DMA] * NBUF,
      [pltpu.SemaphoreType.DMA] * NBUF,
  ]

  @functools.partial(pl.kernel, mesh=mesh, out_type=out_type,
                     scratch_types=scratch,
                     compiler_params=pltpu.CompilerParams(
                         use_tc_tiling_on_sc=False))
  def k(table, src2, dst2, *refs):
    if with_deg:
      out, deg_out, src_a, dst_a, rows, ones_v, acc, dega, gsem, ssem, dsem = refs
    else:
      out, src_a, dst_a, rows, ones_v, acc, dega, gsem, ssem, dsem = refs
      deg_out = None
    c = lax.axis_index("c")
    s = lax.axis_index("s")
    base_r = s * RPT
    srow = c * NP

    pltpu.sync_copy(src2.at[pl.ds(s * NCHUNK, NCHUNK)], src_a)
    pltpu.sync_copy(dst2.at[pl.ds(s * NCHUNK, NCHUNK)], dst_a)

    @pl.when(c == 1)
    def _():
      def _adj(i, _):
        for j in range(C // 16):
          sl = pl.ds(j * 16, 16)
          src_a[i, sl] = src_a[i, sl] + NP
        return 0
      lax.fori_loop(0, NCHUNK, _adj, 0)

    def _zero_rows(i, _):
      for j in range(HW // 16):
        rows[0, i, pl.ds(j * 16, 16)] = jnp.zeros((16,), jnp.float32)
      ones_v[i] = jnp.zeros((DEGW,), jnp.float32)
      return 0
    lax.fori_loop(0, C, _zero_rows, 0)

    for z in range(RPT // C):
      pltpu.sync_copy(rows.at[0], acc.at[pl.ds(base_r + z * C, C)])
      if with_deg:
        pltpu.sync_copy(ones_v, dega.at[pl.ds(base_r + z * C, C)])
    plsc.subcore_barrier()

    if with_deg:
      def _fill_ones(i, _):
        ones_v[i] = jnp.ones((DEGW,), jnp.float32)
        return 0
      lax.fori_loop(0, C, _fill_ones, 0)

    def _fire_gather(g, b):
      pltpu.async_copy(table.at[src_a.at[g]], rows.at[b], gsem[b])

    for b in range(LOOK):
      _fire_gather(b, b)

    def _outer(o, _):
      for b in range(NBUF):
        g = o * NBUF + b
        pltpu.make_async_copy(table.at[src_a.at[g]], rows.at[b],
                              gsem[b]).wait()
        pltpu.async_copy(rows.at[b], acc.at[dst_a.at[g]], ssem[b], add=True)
        if with_deg:
          @pl.when(c == 0)
          def _():
            pltpu.async_copy(ones_v, dega.at[dst_a.at[g]], dsem[b], add=True)
        gn = g + LOOK
        bn = (b + LOOK) % NBUF

        @pl.when(gn < NCHUNK)
        def _():
          @pl.when(gn >= NBUF)
          def _():
            pltpu.make_async_copy(rows.at[bn], acc.at[dst_a.at[gn - NBUF]],
                                  ssem[bn]).wait()
            if with_deg:
              @pl.when(c == 0)
              def _():
                pltpu.make_async_copy(ones_v, dega.at[dst_a.at[gn - NBUF]],
                                      dsem[bn]).wait()
          _fire_gather(gn, bn)
      return 0
    lax.fori_loop(0, NCHUNK // NBUF, _outer, 0)

    for b in range(NBUF):
      g = NCHUNK - NBUF + b
      pltpu.make_async_copy(rows.at[b], acc.at[dst_a.at[g]], ssem[b]).wait()
      if with_deg:
        @pl.when(c == 0)
        def _():
          pltpu.make_async_copy(ones_v, dega.at[dst_a.at[g]], dsem[b]).wait()

    plsc.subcore_barrier()
    pltpu.sync_copy(acc.at[pl.ds(base_r, RPT)],
                    out.at[pl.ds(srow + base_r, RPT)])
    if with_deg:
      @pl.when(c == 0)
      def _():
        pltpu.sync_copy(dega.at[pl.ds(base_r, RPT)],
                        deg_out.at[pl.ds(base_r, RPT)])

  return k


@functools.lru_cache(maxsize=None)
def _sc_agg16():
  mesh = plsc.VectorSubcoreMesh(core_axis_name="c", subcore_axis_name="s")
  out_type = jax.ShapeDtypeStruct((NC * NP, DEGW), jnp.float32)
  NCH = E // (NC * NS) // C
  scratch = [
      pltpu.VMEM((NCH, C), jnp.int32),
      pltpu.VMEM((NCH, C), jnp.int32),
      pltpu.VMEM((NBUF, C, DEGW), jnp.float32),
      pltpu.VMEM_SHARED((NP, DEGW), jnp.float32),
      [pltpu.SemaphoreType.DMA] * NBUF,
      [pltpu.SemaphoreType.DMA] * NBUF,
  ]

  @functools.partial(pl.kernel, mesh=mesh, out_type=out_type,
                     scratch_types=scratch,
                     compiler_params=pltpu.CompilerParams(
                         use_tc_tiling_on_sc=False))
  def k(table, src2, dst2, out, src_a, dst_a, rows, acc, gsem, ssem):
    c = lax.axis_index("c")
    s = lax.axis_index("s")
    t = c * NS + s
    base_r = s * RPT

    pltpu.sync_copy(src2.at[pl.ds(t * NCH, NCH)], src_a)
    pltpu.sync_copy(dst2.at[pl.ds(t * NCH, NCH)], dst_a)

    def _zero_rows(i, _):
      rows[0, i] = jnp.zeros((DEGW,), jnp.float32)
      return 0
    lax.fori_loop(0, C, _zero_rows, 0)
    for z in range(RPT // C):
      pltpu.sync_copy(rows.at[0], acc.at[pl.ds(base_r + z * C, C)])
    plsc.subcore_barrier()

    def _fire_gather(g, b):
      pltpu.async_copy(table.at[src_a.at[g]], rows.at[b], gsem[b])

    for b in range(LOOK):
      _fire_gather(b, b)

    def _outer(o, _):
      for b in range(NBUF):
        g = o * NBUF + b
        pltpu.make_async_copy(table.at[src_a.at[g]], rows.at[b],
                              gsem[b]).wait()
        pltpu.async_copy(rows.at[b], acc.at[dst_a.at[g]], ssem[b], add=True)
        gn = g + LOOK
        bn = (b + LOOK) % NBUF

        @pl.when(gn < NCH)
        def _():
          @pl.when(gn >= NBUF)
          def _():
            pltpu.make_async_copy(rows.at[bn], acc.at[dst_a.at[gn - NBUF]],
                                  ssem[bn]).wait()
          _fire_gather(gn, bn)
      return 0
    lax.fori_loop(0, NCH // NBUF, _outer, 0)

    for b in range(NBUF):
      g = NCH - NBUF + b
      pltpu.make_async_copy(rows.at[b], acc.at[dst_a.at[g]], ssem[b]).wait()

    plsc.subcore_barrier()
    pltpu.sync_copy(acc.at[pl.ds(base_r, RPT)],
                    out.at[pl.ds(c * NP + base_r, RPT)])

  return k


def _tc_prepare(features, W_pre, b_pre2, W_emb1, b_emb1_2, W_emb2, b_emb2_2):
  def body(f_ref, wp_ref, bp_ref, w1_ref, b1_ref, w2_ref, b2_ref,
           table_ref, base_ref):
    h = jnp.dot(f_ref[...], wp_ref[...],
                preferred_element_type=jnp.float32) + bp_ref[...]
    hp = jnp.concatenate([h, jnp.zeros((NP - NT0, HID), jnp.float32)], axis=0)
    table_ref[...] = jnp.concatenate([hp[:, :HW], hp[:, HW:]], axis=0)
    base_ref[...] = jnp.concatenate(
        [h, w1_ref[...] + b1_ref[...], w2_ref[...] + b2_ref[...],
         jnp.zeros((NP - N, HID), jnp.float32)], axis=0)

  return pl.pallas_call(
      body,
      out_shape=(jax.ShapeDtypeStruct((NC * NP, HW), jnp.float32),
                 jax.ShapeDtypeStruct((NP, HID), jnp.float32)),
  )(features, W_pre, b_pre2, W_emb1, b_emb1_2, W_emb2, b_emb2_2)


def _tc_mid(part0, degp, base, assign2, W_ops, W_out_p):
  def body(p_ref, d_ref, b_ref, a_ref, w_ref, wo_ref, ne_ref, z_ref):
    p = p_ref[...]
    acc = jnp.concatenate([p[:NP], p[NP:]], axis=1)
    deg = jnp.maximum(d_ref[...][:, 0:1], 1.0)
    agg0 = acc / deg
    a = a_ref[...]
    w = w_ref[...]
    t = jnp.zeros((NP, HID), jnp.float32)
    for k in range(CLUSTERS):
      mk = (a == k).astype(jnp.float32)
      t = t + mk * jnp.dot(agg0, w[k], preferred_element_type=jnp.float32)
    h = b_ref[...] + t
    ne = jnp.where(h > 0, h, jnp.exp(h) - 1.0)
    ne_ref[...] = ne
    z_ref[...] = jnp.dot(ne, wo_ref[...], preferred_element_type=jnp.float32)

  return pl.pallas_call(
      body,
      out_shape=(jax.ShapeDtypeStruct((NP, HID), jnp.float32),
                 jax.ShapeDtypeStruct((NP, DEGW), jnp.float32)),
  )(part0, degp, base, assign2, W_ops, W_out_p)


def _tc_head(part1, degp, b_out_p2):
  def body(p_ref, d_ref, b_ref, out_ref):
    p = p_ref[...]
    acc = p[:NP] + p[NP:]
    deg = jnp.maximum(d_ref[...][:, 0:1], 1.0)
    out_ref[...] = acc / deg + b_ref[...]

  return pl.pallas_call(
      body,
      out_shape=jax.ShapeDtypeStruct((NP, DEGW), jnp.float32),
  )(part1, degp, b_out_p2)


def kernel(features, edge_index, node_assign, W_pre, b_pre, W_emb1, b_emb1,
           W_emb2, b_emb2, W_ops, W_out, b_out):
  src = edge_index[0].astype(jnp.int32).reshape(E // C, C)
  dst = edge_index[1].astype(jnp.int32).reshape(E // C, C)
  table, base = _tc_prepare(
      features, W_pre, b_pre.reshape(1, HID),
      W_emb1, b_emb1.reshape(1, HID), W_emb2, b_emb2.reshape(1, HID))
  part0, degp = _sc_agg(True)(table, src, dst)
  assign2 = jnp.pad(node_assign.astype(jnp.int32), (0, NP - N),
                    constant_values=-1).reshape(NP, 1)
  W_out_p = jnp.pad(W_out, ((0, 0), (0, DEGW - NUM_CLASSES)))
  ne_p, z = _tc_mid(part0, degp, base, assign2, W_ops, W_out_p)
  part1 = _sc_agg16()(z, src, dst)
  b_out_p = jnp.pad(b_out, (0, DEGW - NUM_CLASSES)).reshape(1, DEGW)
  logits_p = _tc_head(part1, degp, b_out_p)
  return ne_p[:N], logits_p[:N, :NUM_CLASSES]
